# Initial kernel scaffold; baseline (speedup 1.0000x reference)
import jax
import jax.numpy as jnp
from jax.experimental import pallas as pl


def kernel(x, adj, W1, W2, W3):
    raise NotImplementedError("write your pallas kernel here")


# speedup vs baseline: 1.2808x; 1.2808x over previous
import functools

import jax
import jax.numpy as jnp
from jax.experimental import pallas as pl


def _mm(a, b):
    return jax.lax.dot_general(
        a, b, (((1,), (0,)), ((), ())), preferred_element_type=jnp.float32)


def _s1_body(x_ref, w1_ref, s1_ref):
    s1_ref[...] = jnp.tanh(_mm(x_ref[...], w1_ref[...]))


def _p1_body(adj_ref, s1_ref, w2_ref, z1_ref, s2_ref):
    z1 = _mm(adj_ref[...], s1_ref[...])
    z1_ref[...] = z1
    s2_ref[...] = jnp.tanh(_mm(z1, w2_ref[...]))


def _p2_body(adj_ref, z1_ref, s2_ref, w3_ref, az1_ref, z2_ref, cat_ref):
    a = adj_ref[...]
    az1_ref[...] = _mm(a, z1_ref[...])
    z2 = _mm(a, s2_ref[...])
    z2_ref[...] = z2
    cat_ref[...] = jnp.concatenate([z2, _mm(z2, w3_ref[...])], axis=1)


def _p3_body(adj_ref, cat_ref, az2_ref, zi_ref, *, e2):
    t = _mm(adj_ref[...], cat_ref[...])
    az2_ref[...] = t[:, :e2]
    zi_ref[...] = t[:, e2:]


def _p4_body(adj_ref, zi_ref, az3_ref, adjout_ref, *, bm):
    i = pl.program_id(0)
    az3_ref[...] = _mm(adj_ref[...], zi_ref[...])
    zi_i = zi_ref[pl.ds(i * bm, bm), :]
    adjout_ref[...] = jax.nn.sigmoid(jax.lax.dot_general(
        zi_i, zi_ref[...], (((1,), (1,)), ((), ())),
        preferred_element_type=jnp.float32))


def kernel(x, adj, W1, W2, W3):
    n, d_in = x.shape
    e1 = W1.shape[1]
    e2 = W2.shape[1]
    e3 = W3.shape[1]
    f32 = jnp.float32

    bm = 400 if n % 400 == 0 else n
    bm4 = 200 if n % 200 == 0 else n
    ni = n // bm
    ni4 = n // bm4

    s1 = pl.pallas_call(
        _s1_body,
        grid=(ni,),
        in_specs=[pl.BlockSpec((bm, d_in), lambda i: (i, 0)),
                  pl.BlockSpec((d_in, e1), lambda i: (0, 0))],
        out_specs=pl.BlockSpec((bm, e1), lambda i: (i, 0)),
        out_shape=jax.ShapeDtypeStruct((n, e1), f32),
    )(x, W1)

    z1, s2 = pl.pallas_call(
        _p1_body,
        grid=(ni,),
        in_specs=[pl.BlockSpec((bm, n), lambda i: (i, 0)),
                  pl.BlockSpec((n, e1), lambda i: (0, 0)),
                  pl.BlockSpec((e1, e2), lambda i: (0, 0))],
        out_specs=[pl.BlockSpec((bm, e1), lambda i: (i, 0)),
                   pl.BlockSpec((bm, e2), lambda i: (i, 0))],
        out_shape=[jax.ShapeDtypeStruct((n, e1), f32),
                   jax.ShapeDtypeStruct((n, e2), f32)],
    )(adj, s1, W2)

    az1, z2, cat = pl.pallas_call(
        _p2_body,
        grid=(ni,),
        in_specs=[pl.BlockSpec((bm, n), lambda i: (i, 0)),
                  pl.BlockSpec((n, e1), lambda i: (0, 0)),
                  pl.BlockSpec((n, e2), lambda i: (0, 0)),
                  pl.BlockSpec((e2, e3), lambda i: (0, 0))],
        out_specs=[pl.BlockSpec((bm, e1), lambda i: (i, 0)),
                   pl.BlockSpec((bm, e2), lambda i: (i, 0)),
                   pl.BlockSpec((bm, e2 + e3), lambda i: (i, 0))],
        out_shape=[jax.ShapeDtypeStruct((n, e1), f32),
                   jax.ShapeDtypeStruct((n, e2), f32),
                   jax.ShapeDtypeStruct((n, e2 + e3), f32)],
    )(adj, z1, s2, W3)

    az2, zi = pl.pallas_call(
        functools.partial(_p3_body, e2=e2),
        grid=(ni,),
        in_specs=[pl.BlockSpec((bm, n), lambda i: (i, 0)),
                  pl.BlockSpec((n, e2 + e3), lambda i: (0, 0))],
        out_specs=[pl.BlockSpec((bm, e2), lambda i: (i, 0)),
                   pl.BlockSpec((bm, e3), lambda i: (i, 0))],
        out_shape=[jax.ShapeDtypeStruct((n, e2), f32),
                   jax.ShapeDtypeStruct((n, e3), f32)],
    )(adj, cat)

    az3, adjout = pl.pallas_call(
        functools.partial(_p4_body, bm=bm4),
        grid=(ni4,),
        in_specs=[pl.BlockSpec((bm4, n), lambda i: (i, 0)),
                  pl.BlockSpec((n, e3), lambda i: (0, 0))],
        out_specs=[pl.BlockSpec((bm4, e3), lambda i: (i, 0)),
                   pl.BlockSpec((bm4, n), lambda i: (i, 0))],
        out_shape=[jax.ShapeDtypeStruct((n, e3), f32),
                   jax.ShapeDtypeStruct((n, n), f32)],
    )(adj, zi)

    return (zi, adjout, az1, az2, az3, z1, z2)
